# baseline (speedup 1.0000x reference)
import functools

import jax
import jax.numpy as jnp
from jax import lax
from jax.experimental import pallas as pl
from jax.experimental.pallas import tpu as pltpu
from jax.experimental.pallas import tpu_sc as plsc

B, N_MAX, C = 2, 16384, 256
G, K = 512, 64
H = 8
HD = C // H
SCALE = HD ** -0.5

BG = B * G
R = B * G * K
RB = 128
NW = 32
ROWS_PER_W = R // NW
CW = 256
CNT_PAD = 256

NBLK = ROWS_PER_W // RB


def _gather_body(idx_hbm, feats_hbm, out_hbm, idx_v, rows0, rows1, sem0, sem1):
    cid = lax.axis_index("c")
    sid = lax.axis_index("s")
    wid = sid * 2 + cid
    base = wid * ROWS_PER_W
    off = jnp.where(base >= R // 2, N_MAX, 0).astype(jnp.int32)

    pltpu.sync_copy(idx_hbm.at[pl.ds(base, ROWS_PER_W)], idx_v)

    def addoff(i, _):
        idx_v[pl.ds(i * 16, 16)] = idx_v[pl.ds(i * 16, 16)] + off
        return 0

    lax.fori_loop(0, ROWS_PER_W // 16, addoff, 0)

    bufs = (rows0, rows1)
    sems = (sem0, sem1)

    pltpu.async_copy(feats_hbm.at[idx_v.at[pl.ds(0, RB)]], bufs[0], sems[0])

    def body(i, _):
        for b in range(2):
            m = 2 * i + b
            nb = 1 - b

            @pl.when(m + 1 < NBLK)
            def _():
                o = pl.multiple_of((m + 1) * RB, RB)
                pltpu.async_copy(
                    feats_hbm.at[idx_v.at[pl.ds(o, RB)]], bufs[nb], sems[nb])

            pltpu.make_async_copy(
                feats_hbm.at[idx_v.at[pl.ds(0, RB)]], bufs[b],
                sems[b]).wait()
            pltpu.sync_copy(bufs[b], out_hbm.at[pl.ds(base + m * RB, RB)])
        return 0

    lax.fori_loop(0, NBLK // 2, body, 0)


def _gather_sc(idx_flat, feats_flat):
    mesh = plsc.VectorSubcoreMesh(core_axis_name="c", subcore_axis_name="s")
    fn = pl.kernel(
        _gather_body,
        mesh=mesh,
        out_type=jax.ShapeDtypeStruct((R, C), jnp.float32),
        scratch_types=[
            pltpu.VMEM((ROWS_PER_W,), jnp.int32),
            pltpu.VMEM((RB, C), jnp.float32),
            pltpu.VMEM((RB, C), jnp.float32),
            pltpu.SemaphoreType.DMA,
            pltpu.SemaphoreType.DMA,
        ],
    )
    return fn(idx_flat, feats_flat)


GPB = 64


def _attn_body(xg_ref, mb_ref, mcol_ref, wqkv_ref, wproj_ref, bproj_ref,
               out_ref):
    f32 = jnp.float32
    bf16 = jnp.bfloat16

    r_i = lax.broadcasted_iota(jnp.int32, (H * K, C), 0)
    c_i = lax.broadcasted_iota(jnp.int32, (H * K, C), 1)
    M = (r_i // K == c_i // HD).astype(bf16)
    br = lax.broadcasted_iota(jnp.int32, (H * K, H), 0)
    bc = lax.broadcasted_iota(jnp.int32, (H * K, H), 1)
    blk = (br // K == bc).astype(bf16)
    di = lax.broadcasted_iota(jnp.int32, (K, K), 0)
    dj = lax.broadcasted_iota(jnp.int32, (K, K), 1)
    eye = (di == dj).astype(bf16)

    hc_i = lax.broadcasted_iota(jnp.int32, (H, C), 0)
    hc_c = lax.broadcasted_iota(jnp.int32, (H, C), 1)
    blkC = (hc_i == hc_c // HD).astype(bf16)

    X = xg_ref[...].astype(bf16)
    qkv = jnp.dot(X, wqkv_ref[...], preferred_element_type=f32)
    bias = jnp.broadcast_to(mb_ref[...], (GPB, K, H * K)).reshape(GPB * K, H * K)

    logits = []
    for t in range(GPB):
        sl = slice(t * K, (t + 1) * K)
        q = (qkv[sl, :C] * SCALE).astype(bf16)
        k = qkv[sl, C:2 * C].astype(bf16)
        K2 = jnp.concatenate([k] * H, axis=0) * M
        logits.append(lax.dot_general(q, K2, (((1,), (1,)), ((), ())),
                                      preferred_element_type=f32))
    logits = jnp.concatenate(logits, axis=0)
    e = jnp.exp(logits + bias)
    eb = e.astype(bf16)

    s = jnp.dot(eb, blk, preferred_element_type=f32)
    r = (1.0 / jnp.maximum(s, 1e-30)).astype(bf16)
    rav = jnp.dot(r, blkC, preferred_element_type=f32)

    av0 = []
    for t in range(GPB):
        sl = slice(t * K, (t + 1) * K)
        v = qkv[sl, 2 * C:].astype(bf16)
        V2 = jnp.concatenate([v] * H, axis=0) * M
        av0.append(jnp.dot(eb[sl], V2, preferred_element_type=f32))
    av = (jnp.concatenate(av0, axis=0) * rav).astype(bf16)

    out = jnp.dot(av, wproj_ref[...], preferred_element_type=f32) \
        + bproj_ref[...]
    out_ref[...] = out * mcol_ref[...]


def _attn_tc(xg, mask_bias, mask_col, W_qkv, W_proj, b_proj2):
    return pl.pallas_call(
        _attn_body,
        grid=(BG // GPB,),
        in_specs=[
            pl.BlockSpec((GPB * K, C), lambda g: (g, 0)),
            pl.BlockSpec((GPB, 1, H * K), lambda g: (g, 0, 0)),
            pl.BlockSpec((GPB * K, 1), lambda g: (g, 0)),
            pl.BlockSpec((C, 3 * C), lambda g: (0, 0)),
            pl.BlockSpec((C, C), lambda g: (0, 0)),
            pl.BlockSpec((1, C), lambda g: (0, 0)),
        ],
        out_specs=pl.BlockSpec((GPB * K, C), lambda g: (g, 0)),
        out_shape=jax.ShapeDtypeStruct((R, C), jnp.float32),
    )(xg, mask_bias, mask_col, W_qkv, W_proj, b_proj2)


def _scatter_body(idx_hbm, mask_hbm, upd_hbm,
                  acc_out, cnt_out,
                  idx2d, tgt2d, mask2d, rows_v, rows_v2, ones_v, sem0, sem1):
    cid = lax.axis_index("c")
    sid = lax.axis_index("s")
    rows_per_batch = R // B

    def zinit(i, _):
        for j in range(C // 16):
            rows_v[i, pl.ds(j * 16, 16)] = jnp.zeros((16,), jnp.float32)
        for j in range(CW // 16):
            ones_v[i, pl.ds(j * 16, 16)] = jnp.zeros((16,), jnp.float32)
        return 0

    HW = 128

    lax.fori_loop(0, RB, zinit, 0)
    zbase = cid * N_MAX + sid * (N_MAX // 16)

    def zslab(s, _):
        pltpu.sync_copy(rows_v, acc_out.at[pl.ds(zbase + s * RB, RB)])
        pltpu.sync_copy(ones_v.at[:, pl.ds(0, HW)],
                        cnt_out.at[pl.ds(zbase + s * RB, RB), pl.ds(0, HW)])
        return 0

    lax.fori_loop(0, N_MAX // 16 // RB, zslab, 0)
    dump0 = B * N_MAX + cid * 128

    @pl.when(sid == 0)
    def _():
        pltpu.sync_copy(ones_v.at[:, pl.ds(0, HW)],
                        cnt_out.at[pl.ds(dump0, RB), pl.ds(0, HW)])

    def oinit(i, _):
        for j in range(CW // 16):
            ones_v[i, pl.ds(j * 16, 16)] = jnp.ones((16,), jnp.float32)
        return 0

    lax.fori_loop(0, RB, oinit, 0)
    plsc.subcore_barrier()

    rb16 = pl.multiple_of(
        (cid * rows_per_batch + sid * (rows_per_batch // 16)) // RB, NBLK)
    tgt_off = cid * N_MAX
    lane = lax.iota(jnp.int32, 16)
    pltpu.sync_copy(idx_hbm.at[pl.ds(rb16, NBLK)], idx2d)
    pltpu.sync_copy(mask_hbm.at[pl.ds(rb16, NBLK)], mask2d)

    def tbody(blk, _):
        for j in range(RB // 16):
            v = idx2d[blk, pl.ds(j * 16, 16)] + tgt_off
            m = mask2d[blk, pl.ds(j * 16, 16)]
            idx2d[blk, pl.ds(j * 16, 16)] = v
            tgt2d[blk, pl.ds(j * 16, 16)] = jnp.where(
                m > 0, v, dump0 + j * 16 + lane)
        return 0

    lax.fori_loop(0, NBLK, tbody, 0)

    bufs = (rows_v, rows_v2)
    sems = (sem0, sem1)
    row0 = rb16 * RB
    pltpu.async_copy(upd_hbm.at[pl.ds(row0, RB)], bufs[0], sems[0])

    def sbody(i, _):
        for b in range(2):
            m = 2 * i + b
            nb = 1 - b

            @pl.when(m + 1 < NBLK)
            def _():
                o = pl.multiple_of(row0 + (m + 1) * RB, RB)
                pltpu.async_copy(upd_hbm.at[pl.ds(o, RB)], bufs[nb], sems[nb])

            pltpu.make_async_copy(
                upd_hbm.at[pl.ds(row0, RB)], bufs[b], sems[b]).wait()
            pltpu.sync_copy(bufs[b], acc_out.at[idx2d.at[m]], add=True)
            pltpu.sync_copy(ones_v.at[:, pl.ds(0, HW)],
                            cnt_out.at[tgt2d.at[m], pl.ds(0, HW)], add=True)
        return 0

    lax.fori_loop(0, NBLK // 2, sbody, 0)


def _scatter_sc(idx2d_in, maskf2d, updated):
    mesh = plsc.VectorSubcoreMesh(core_axis_name="c", subcore_axis_name="s")
    fn = pl.kernel(
        _scatter_body,
        mesh=mesh,
        out_type=(
            jax.ShapeDtypeStruct((B * N_MAX, C), jnp.float32),
            jax.ShapeDtypeStruct((B * N_MAX + CNT_PAD, CW), jnp.float32),
        ),
        scratch_types=[
            pltpu.VMEM((NBLK, RB), jnp.int32),
            pltpu.VMEM((NBLK, RB), jnp.int32),
            pltpu.VMEM((NBLK, RB), jnp.float32),
            pltpu.VMEM((RB, C), jnp.float32),
            pltpu.VMEM((RB, C), jnp.float32),
            pltpu.VMEM((RB, CW), jnp.float32),
            pltpu.SemaphoreType.DMA,
            pltpu.SemaphoreType.DMA,
        ],
    )
    return fn(idx2d_in, maskf2d, updated)


def _combine_body(up_ref, acc_ref, cnt_ref, gamma_ref, out_ref):
    cnt = jnp.max(cnt_ref[...], axis=1, keepdims=True)
    denom = jnp.maximum(cnt, 1.0)
    out_ref[...] = up_ref[...] + acc_ref[...] * gamma_ref[...] / denom


def _combine_tc(up, acc, cnt, gamma2):
    BLK = 512
    return pl.pallas_call(
        _combine_body,
        grid=(B * N_MAX // BLK,),
        in_specs=[
            pl.BlockSpec((BLK, C), lambda g: (g, 0)),
            pl.BlockSpec((BLK, C), lambda g: (g, 0)),
            pl.BlockSpec((BLK, 128), lambda g: (g, 0)),
        pl.BlockSpec((1, C), lambda g: (0, 0)),
        ],
        out_specs=pl.BlockSpec((BLK, C), lambda g: (g, 0)),
        out_shape=jax.ShapeDtypeStruct((B * N_MAX, C), jnp.float32),
    )(up, acc, cnt, gamma2)


def kernel(upscaled_feats, grouping_idx, grouping_point_mask, W_qkv, W_proj,
           b_proj, gamma):
    idx = jnp.where(grouping_idx < 0, 0, grouping_idx).astype(jnp.int32)
    idx_flat = idx.reshape(R)
    idx2d_in = idx.reshape(R // RB, RB)
    feats_flat = upscaled_feats.reshape(B * N_MAX, C)

    xg = _gather_sc(idx_flat, feats_flat)

    maskf = grouping_point_mask.astype(jnp.float32)
    mask_bias = jnp.tile((maskf.reshape(BG, 1, K) - 1.0) * 1e5, (1, 1, H))
    updated = _attn_tc(xg, mask_bias, maskf.reshape(R, 1),
                       W_qkv.astype(jnp.bfloat16),
                       W_proj.astype(jnp.bfloat16), b_proj.reshape(1, C))

    acc, cntp = _scatter_sc(idx2d_in, maskf.reshape(R // RB, RB), updated)

    out = _combine_tc(feats_flat, acc, cntp, gamma.reshape(1, C))
    return out.reshape(B, N_MAX, C)


# speedup vs baseline: 1.0318x; 1.0318x over previous
import functools

import jax
import jax.numpy as jnp
from jax import lax
from jax.experimental import pallas as pl
from jax.experimental.pallas import tpu as pltpu
from jax.experimental.pallas import tpu_sc as plsc

B, N_MAX, C = 2, 16384, 256
G, K = 512, 64
H = 8
HD = C // H
SCALE = HD ** -0.5

BG = B * G
R = B * G * K
RB = 128
NW = 32
ROWS_PER_W = R // NW
CW = 256
CNT_PAD = 256

NBLK = ROWS_PER_W // RB


def _gather_body(idx_hbm, feats_hbm, out_hbm, idx_v, rows0, rows1, sem0, sem1):
    cid = lax.axis_index("c")
    sid = lax.axis_index("s")
    wid = sid * 2 + cid
    base = wid * ROWS_PER_W
    off = jnp.where(base >= R // 2, N_MAX, 0).astype(jnp.int32)

    pltpu.sync_copy(idx_hbm.at[pl.ds(base, ROWS_PER_W)], idx_v)

    def addoff(i, _):
        idx_v[pl.ds(i * 16, 16)] = idx_v[pl.ds(i * 16, 16)] + off
        return 0

    lax.fori_loop(0, ROWS_PER_W // 16, addoff, 0)

    bufs = (rows0, rows1)
    sems = (sem0, sem1)

    pltpu.async_copy(feats_hbm.at[idx_v.at[pl.ds(0, RB)]], bufs[0], sems[0])

    def body(i, _):
        for b in range(2):
            m = 2 * i + b
            nb = 1 - b

            @pl.when(m + 1 < NBLK)
            def _():
                o = pl.multiple_of((m + 1) * RB, RB)
                pltpu.async_copy(
                    feats_hbm.at[idx_v.at[pl.ds(o, RB)]], bufs[nb], sems[nb])

            pltpu.make_async_copy(
                feats_hbm.at[idx_v.at[pl.ds(0, RB)]], bufs[b],
                sems[b]).wait()
            pltpu.sync_copy(bufs[b], out_hbm.at[pl.ds(base + m * RB, RB)])
        return 0

    lax.fori_loop(0, NBLK // 2, body, 0)


def _gather_sc(idx_flat, feats_flat):
    mesh = plsc.VectorSubcoreMesh(core_axis_name="c", subcore_axis_name="s")
    fn = pl.kernel(
        _gather_body,
        mesh=mesh,
        out_type=jax.ShapeDtypeStruct((R, C), jnp.float32),
        scratch_types=[
            pltpu.VMEM((ROWS_PER_W,), jnp.int32),
            pltpu.VMEM((RB, C), jnp.float32),
            pltpu.VMEM((RB, C), jnp.float32),
            pltpu.SemaphoreType.DMA,
            pltpu.SemaphoreType.DMA,
        ],
    )
    return fn(idx_flat, feats_flat)


GPB = 64


def _attn_body(xg_ref, mb_ref, wqkv_ref, wproj_ref, bproj_ref, out_ref):
    f32 = jnp.float32
    bf16 = jnp.bfloat16

    r_i = lax.broadcasted_iota(jnp.int32, (H * K, C), 0)
    c_i = lax.broadcasted_iota(jnp.int32, (H * K, C), 1)
    M = (r_i // K == c_i // HD).astype(bf16)
    br = lax.broadcasted_iota(jnp.int32, (H * K, H), 0)
    bc = lax.broadcasted_iota(jnp.int32, (H * K, H), 1)
    blk = (br // K == bc).astype(bf16)
    di = lax.broadcasted_iota(jnp.int32, (K, K), 0)
    dj = lax.broadcasted_iota(jnp.int32, (K, K), 1)
    eye = (di == dj).astype(bf16)

    hc_i = lax.broadcasted_iota(jnp.int32, (H, C), 0)
    hc_c = lax.broadcasted_iota(jnp.int32, (H, C), 1)
    blkC = (hc_i == hc_c // HD).astype(bf16)

    X = xg_ref[...].astype(bf16)
    qkv = jnp.dot(X, wqkv_ref[...], preferred_element_type=f32)
    bias = jnp.broadcast_to(mb_ref[...], (GPB, K, H * K)).reshape(GPB * K, H * K)

    logits = []
    for t in range(GPB):
        sl = slice(t * K, (t + 1) * K)
        q = (qkv[sl, :C] * SCALE).astype(bf16)
        k = qkv[sl, C:2 * C].astype(bf16)
        K2 = jnp.concatenate([k] * H, axis=0) * M
        logits.append(lax.dot_general(q, K2, (((1,), (1,)), ((), ())),
                                      preferred_element_type=f32))
    logits = jnp.concatenate(logits, axis=0)
    e = jnp.exp(logits + bias)
    eb = e.astype(bf16)

    s = jnp.dot(eb, blk, preferred_element_type=f32)
    r = (1.0 / jnp.maximum(s, 1e-30)).astype(bf16)
    rav = jnp.dot(r, blkC, preferred_element_type=f32)

    av0 = []
    for t in range(GPB):
        sl = slice(t * K, (t + 1) * K)
        v = qkv[sl, 2 * C:].astype(bf16)
        V2 = jnp.concatenate([v] * H, axis=0) * M
        av0.append(jnp.dot(eb[sl], V2, preferred_element_type=f32))
    av = (jnp.concatenate(av0, axis=0) * rav).astype(bf16)

    out = jnp.dot(av, wproj_ref[...], preferred_element_type=f32) \
        + bproj_ref[...]
    outb = out.astype(bf16)
    for t in range(GPB):
        m01 = jnp.where(mb_ref[t][:, :K] < -1.0, 0.0, 1.0).astype(bf16)
        D = eye * m01
        out_ref[t * K:(t + 1) * K, :] = jnp.dot(
            D, outb[t * K:(t + 1) * K, :], preferred_element_type=f32)


def _attn_tc(xg, mask_bias, W_qkv, W_proj, b_proj2):
    return pl.pallas_call(
        _attn_body,
        grid=(BG // GPB,),
        in_specs=[
            pl.BlockSpec((GPB * K, C), lambda g: (g, 0)),
            pl.BlockSpec((GPB, 1, H * K), lambda g: (g, 0, 0)),
            pl.BlockSpec((C, 3 * C), lambda g: (0, 0)),
            pl.BlockSpec((C, C), lambda g: (0, 0)),
            pl.BlockSpec((1, C), lambda g: (0, 0)),
        ],
        out_specs=pl.BlockSpec((GPB * K, C), lambda g: (g, 0)),
        out_shape=jax.ShapeDtypeStruct((R, C), jnp.float32),
    )(xg, mask_bias, W_qkv, W_proj, b_proj2)


def _scatter_body(idx_hbm, mask_hbm, upd_hbm,
                  acc_out, cnt_out,
                  idx2d, tgt2d, mask2d, rows_v, rows_v2, ones_v, sem0, sem1):
    cid = lax.axis_index("c")
    sid = lax.axis_index("s")
    rows_per_batch = R // B

    def zinit(i, _):
        for j in range(C // 16):
            rows_v[i, pl.ds(j * 16, 16)] = jnp.zeros((16,), jnp.float32)
        for j in range(CW // 16):
            ones_v[i, pl.ds(j * 16, 16)] = jnp.zeros((16,), jnp.float32)
        return 0

    HW = 128

    lax.fori_loop(0, RB, zinit, 0)
    zbase = cid * N_MAX + sid * (N_MAX // 16)

    def zslab(s, _):
        pltpu.sync_copy(rows_v, acc_out.at[pl.ds(zbase + s * RB, RB)])
        pltpu.sync_copy(ones_v.at[:, pl.ds(0, HW)],
                        cnt_out.at[pl.ds(zbase + s * RB, RB), pl.ds(0, HW)])
        return 0

    lax.fori_loop(0, N_MAX // 16 // RB, zslab, 0)
    dump0 = B * N_MAX + cid * 128

    @pl.when(sid == 0)
    def _():
        pltpu.sync_copy(ones_v.at[:, pl.ds(0, HW)],
                        cnt_out.at[pl.ds(dump0, RB), pl.ds(0, HW)])

    def oinit(i, _):
        for j in range(CW // 16):
            ones_v[i, pl.ds(j * 16, 16)] = jnp.ones((16,), jnp.float32)
        return 0

    lax.fori_loop(0, RB, oinit, 0)
    plsc.subcore_barrier()

    rb16 = pl.multiple_of(
        (cid * rows_per_batch + sid * (rows_per_batch // 16)) // RB, NBLK)
    tgt_off = cid * N_MAX
    lane = lax.iota(jnp.int32, 16)
    pltpu.sync_copy(idx_hbm.at[pl.ds(rb16, NBLK)], idx2d)
    pltpu.sync_copy(mask_hbm.at[pl.ds(rb16, NBLK)], mask2d)

    def tbody(blk, _):
        for j in range(RB // 16):
            v = idx2d[blk, pl.ds(j * 16, 16)] + tgt_off
            m = mask2d[blk, pl.ds(j * 16, 16)]
            idx2d[blk, pl.ds(j * 16, 16)] = v
            tgt2d[blk, pl.ds(j * 16, 16)] = jnp.where(
                m > 0, v, dump0 + j * 16 + lane)
        return 0

    lax.fori_loop(0, NBLK, tbody, 0)

    bufs = (rows_v, rows_v2)
    sems = (sem0, sem1)
    row0 = rb16 * RB
    pltpu.async_copy(upd_hbm.at[pl.ds(row0, RB)], bufs[0], sems[0])

    def sbody(i, _):
        for b in range(2):
            m = 2 * i + b
            nb = 1 - b

            @pl.when(m + 1 < NBLK)
            def _():
                o = pl.multiple_of(row0 + (m + 1) * RB, RB)
                pltpu.async_copy(upd_hbm.at[pl.ds(o, RB)], bufs[nb], sems[nb])

            pltpu.make_async_copy(
                upd_hbm.at[pl.ds(row0, RB)], bufs[b], sems[b]).wait()
            pltpu.sync_copy(bufs[b], acc_out.at[idx2d.at[m]], add=True)
            pltpu.sync_copy(ones_v.at[:, pl.ds(0, HW)],
                            cnt_out.at[tgt2d.at[m], pl.ds(0, HW)], add=True)
        return 0

    lax.fori_loop(0, NBLK // 2, sbody, 0)


def _scatter_sc(idx2d_in, maskf2d, updated):
    mesh = plsc.VectorSubcoreMesh(core_axis_name="c", subcore_axis_name="s")
    fn = pl.kernel(
        _scatter_body,
        mesh=mesh,
        out_type=(
            jax.ShapeDtypeStruct((B * N_MAX, C), jnp.float32),
            jax.ShapeDtypeStruct((B * N_MAX + CNT_PAD, CW), jnp.float32),
        ),
        scratch_types=[
            pltpu.VMEM((NBLK, RB), jnp.int32),
            pltpu.VMEM((NBLK, RB), jnp.int32),
            pltpu.VMEM((NBLK, RB), jnp.float32),
            pltpu.VMEM((RB, C), jnp.float32),
            pltpu.VMEM((RB, C), jnp.float32),
            pltpu.VMEM((RB, CW), jnp.float32),
            pltpu.SemaphoreType.DMA,
            pltpu.SemaphoreType.DMA,
        ],
    )
    return fn(idx2d_in, maskf2d, updated)


def _combine_body(up_ref, acc_ref, cnt_ref, gamma_ref, out_ref):
    cnt = jnp.max(cnt_ref[...], axis=1, keepdims=True)
    denom = jnp.maximum(cnt, 1.0)
    out_ref[...] = up_ref[...] + acc_ref[...] * gamma_ref[...] / denom


def _combine_tc(up, acc, cnt, gamma2):
    BLK = 512
    return pl.pallas_call(
        _combine_body,
        grid=(B * N_MAX // BLK,),
        in_specs=[
            pl.BlockSpec((BLK, C), lambda g: (g, 0)),
            pl.BlockSpec((BLK, C), lambda g: (g, 0)),
            pl.BlockSpec((BLK, 128), lambda g: (g, 0)),
        pl.BlockSpec((1, C), lambda g: (0, 0)),
        ],
        out_specs=pl.BlockSpec((BLK, C), lambda g: (g, 0)),
        out_shape=jax.ShapeDtypeStruct((B * N_MAX, C), jnp.float32),
    )(up, acc, cnt, gamma2)


def kernel(upscaled_feats, grouping_idx, grouping_point_mask, W_qkv, W_proj,
           b_proj, gamma):
    idx = jnp.where(grouping_idx < 0, 0, grouping_idx).astype(jnp.int32)
    idx_flat = idx.reshape(R)
    idx2d_in = idx.reshape(R // RB, RB)
    feats_flat = upscaled_feats.reshape(B * N_MAX, C)

    xg = _gather_sc(idx_flat, feats_flat)

    maskf = grouping_point_mask.astype(jnp.float32)
    mask_bias = jnp.tile((maskf.reshape(BG, 1, K) - 1.0) * 1e5, (1, 1, H))
    updated = _attn_tc(xg, mask_bias, W_qkv.astype(jnp.bfloat16),
                       W_proj.astype(jnp.bfloat16), b_proj.reshape(1, C))

    acc, cntp = _scatter_sc(idx2d_in, maskf.reshape(R // RB, RB), updated)

    out = _combine_tc(feats_flat, acc, cntp, gamma.reshape(1, C))
    return out.reshape(B, N_MAX, C)


# speedup vs baseline: 1.0895x; 1.0559x over previous
import functools

import jax
import jax.numpy as jnp
from jax import lax
from jax.experimental import pallas as pl
from jax.experimental.pallas import tpu as pltpu
from jax.experimental.pallas import tpu_sc as plsc

B, N_MAX, C = 2, 16384, 256
G, K = 512, 64
H = 8
HD = C // H
SCALE = HD ** -0.5

BG = B * G
R = B * G * K
RB = 128
NW = 32
ROWS_PER_W = R // NW
CW = 256
CNT_PAD = 256

NBLK = ROWS_PER_W // RB


def _gather_body(idx_hbm, feats_hbm, out_hbm, idx_v, rows0, rows1, sem0, sem1):
    cid = lax.axis_index("c")
    sid = lax.axis_index("s")
    wid = sid * 2 + cid
    base = wid * ROWS_PER_W
    off = jnp.where(base >= R // 2, N_MAX, 0).astype(jnp.int32)

    pltpu.sync_copy(idx_hbm.at[pl.ds(base, ROWS_PER_W)], idx_v)

    def addoff(i, _):
        idx_v[pl.ds(i * 16, 16)] = idx_v[pl.ds(i * 16, 16)] + off
        return 0

    lax.fori_loop(0, ROWS_PER_W // 16, addoff, 0)

    bufs = (rows0, rows1)
    sems = (sem0, sem1)

    pltpu.async_copy(feats_hbm.at[idx_v.at[pl.ds(0, RB)]], bufs[0], sems[0])

    def body(i, _):
        for b in range(2):
            m = 2 * i + b
            nb = 1 - b

            @pl.when(m + 1 < NBLK)
            def _():
                o = pl.multiple_of((m + 1) * RB, RB)
                pltpu.async_copy(
                    feats_hbm.at[idx_v.at[pl.ds(o, RB)]], bufs[nb], sems[nb])

            pltpu.make_async_copy(
                feats_hbm.at[idx_v.at[pl.ds(0, RB)]], bufs[b],
                sems[b]).wait()
            pltpu.sync_copy(bufs[b], out_hbm.at[pl.ds(base + m * RB, RB)])
        return 0

    lax.fori_loop(0, NBLK // 2, body, 0)


def _gather_sc(idx_flat, feats_flat):
    mesh = plsc.VectorSubcoreMesh(core_axis_name="c", subcore_axis_name="s")
    fn = pl.kernel(
        _gather_body,
        mesh=mesh,
        out_type=jax.ShapeDtypeStruct((R, C), jnp.float32),
        scratch_types=[
            pltpu.VMEM((ROWS_PER_W,), jnp.int32),
            pltpu.VMEM((RB, C), jnp.float32),
            pltpu.VMEM((RB, C), jnp.float32),
            pltpu.SemaphoreType.DMA,
            pltpu.SemaphoreType.DMA,
        ],
    )
    return fn(idx_flat, feats_flat)


GPB = 64


def _attn_body(xg_ref, mb_ref, wqkv_ref, wproj_ref, bproj_ref, out_ref):
    f32 = jnp.float32
    bf16 = jnp.bfloat16

    r_i = lax.broadcasted_iota(jnp.int32, (H * K, C), 0)
    c_i = lax.broadcasted_iota(jnp.int32, (H * K, C), 1)
    M = (r_i // K == c_i // HD).astype(bf16)
    br = lax.broadcasted_iota(jnp.int32, (H * K, H), 0)
    bc = lax.broadcasted_iota(jnp.int32, (H * K, H), 1)
    blk = (br // K == bc).astype(bf16)
    di = lax.broadcasted_iota(jnp.int32, (K, K), 0)
    dj = lax.broadcasted_iota(jnp.int32, (K, K), 1)
    eye = (di == dj).astype(bf16)

    hc_i = lax.broadcasted_iota(jnp.int32, (H, C), 0)
    hc_c = lax.broadcasted_iota(jnp.int32, (H, C), 1)
    blkC = (hc_i == hc_c // HD).astype(bf16)

    X = xg_ref[...].astype(bf16)
    qkv = jnp.dot(X, wqkv_ref[...], preferred_element_type=f32)
    bias = jnp.broadcast_to(mb_ref[...], (GPB, K, H * K)).reshape(GPB * K, H * K)

    logits = []
    for t in range(GPB):
        sl = slice(t * K, (t + 1) * K)
        q = (qkv[sl, :C] * SCALE).astype(bf16)
        k = qkv[sl, C:2 * C].astype(bf16)
        K2 = jnp.concatenate([k] * H, axis=0) * M
        logits.append(lax.dot_general(q, K2, (((1,), (1,)), ((), ())),
                                      preferred_element_type=f32))
    logits = jnp.concatenate(logits, axis=0)
    e = jnp.exp(logits + bias)
    eb = e.astype(bf16)

    s = jnp.dot(eb, blk, preferred_element_type=f32)
    r = (1.0 / jnp.maximum(s, 1e-30)).astype(bf16)
    rav = jnp.dot(r, blkC, preferred_element_type=f32)

    av0 = []
    for t in range(GPB):
        sl = slice(t * K, (t + 1) * K)
        v = qkv[sl, 2 * C:].astype(bf16)
        V2 = jnp.concatenate([v] * H, axis=0) * M
        av0.append(jnp.dot(eb[sl], V2, preferred_element_type=f32))
    av = (jnp.concatenate(av0, axis=0) * rav).astype(bf16)

    out = jnp.dot(av, wproj_ref[...], preferred_element_type=f32) \
        + bproj_ref[...]
    outb = out.astype(bf16)
    for t in range(GPB):
        m01 = jnp.where(mb_ref[t][:, :K] < -1.0, 0.0, 1.0).astype(bf16)
        D = eye * m01
        out_ref[t * K:(t + 1) * K, :] = jnp.dot(
            D, outb[t * K:(t + 1) * K, :], preferred_element_type=f32)


def _attn_tc(xg, mask_bias, W_qkv, W_proj, b_proj2):
    return pl.pallas_call(
        _attn_body,
        grid=(BG // GPB,),
        in_specs=[
            pl.BlockSpec((GPB * K, C), lambda g: (g, 0)),
            pl.BlockSpec((GPB, 1, H * K), lambda g: (g, 0, 0)),
            pl.BlockSpec((C, 3 * C), lambda g: (0, 0)),
            pl.BlockSpec((C, C), lambda g: (0, 0)),
            pl.BlockSpec((1, C), lambda g: (0, 0)),
        ],
        out_specs=pl.BlockSpec((GPB * K, C), lambda g: (g, 0)),
        out_shape=jax.ShapeDtypeStruct((R, C), jnp.float32),
    )(xg, mask_bias, W_qkv, W_proj, b_proj2)


def _scatter_body(idx_hbm, mask_hbm, upd_hbm,
                  acc_out, cnt_out,
                  idx2d, tgt2d, mask2d, rows_v, rows_v2, ones_v, sem0, sem1):
    cid = lax.axis_index("c")
    sid = lax.axis_index("s")
    rows_per_batch = R // B

    def zinit(i, _):
        for j in range(C // 16):
            rows_v[i, pl.ds(j * 16, 16)] = jnp.zeros((16,), jnp.float32)
        for j in range(CW // 16):
            ones_v[i, pl.ds(j * 16, 16)] = jnp.zeros((16,), jnp.float32)
        return 0

    HW = 128

    lax.fori_loop(0, RB, zinit, 0)
    zbase = cid * N_MAX + sid * (N_MAX // 16)

    def zslab(s, _):
        pltpu.sync_copy(rows_v, acc_out.at[pl.ds(zbase + s * RB, RB)])
        pltpu.sync_copy(ones_v.at[:, pl.ds(0, HW)],
                        cnt_out.at[pl.ds(zbase + s * RB, RB), pl.ds(0, HW)])
        return 0

    lax.fori_loop(0, N_MAX // 16 // RB, zslab, 0)
    dump0 = B * N_MAX + cid * 128

    @pl.when(sid == 0)
    def _():
        pltpu.sync_copy(ones_v.at[:, pl.ds(0, HW)],
                        cnt_out.at[pl.ds(dump0, RB), pl.ds(0, HW)])

    def oinit(i, _):
        for j in range(CW // 16):
            ones_v[i, pl.ds(j * 16, 16)] = jnp.ones((16,), jnp.float32)
        return 0

    lax.fori_loop(0, RB, oinit, 0)
    plsc.subcore_barrier()

    rb16 = pl.multiple_of(
        (cid * rows_per_batch + sid * (rows_per_batch // 16)) // RB, NBLK)
    tgt_off = cid * N_MAX
    lane = lax.iota(jnp.int32, 16)
    pltpu.sync_copy(idx_hbm.at[pl.ds(rb16, NBLK)], idx2d)
    pltpu.sync_copy(mask_hbm.at[pl.ds(rb16, NBLK)], mask2d)

    def tbody(blk, _):
        for j in range(RB // 16):
            v = idx2d[blk, pl.ds(j * 16, 16)] + tgt_off
            m = mask2d[blk, pl.ds(j * 16, 16)]
            idx2d[blk, pl.ds(j * 16, 16)] = v
            tgt2d[blk, pl.ds(j * 16, 16)] = jnp.where(
                m > 0, v, dump0 + j * 16 + lane)
        return 0

    lax.fori_loop(0, NBLK, tbody, 0)

    bufs = (rows_v, rows_v2)
    sems = (sem0, sem1)
    row0 = rb16 * RB
    pltpu.async_copy(upd_hbm.at[pl.ds(row0, RB)], bufs[0], sems[0])

    def sbody(i, _):
        for b in range(2):
            m = 2 * i + b
            nb = 1 - b

            @pl.when(m + 1 < NBLK)
            def _():
                o = pl.multiple_of(row0 + (m + 1) * RB, RB)
                pltpu.async_copy(upd_hbm.at[pl.ds(o, RB)], bufs[nb], sems[nb])

            pltpu.make_async_copy(
                upd_hbm.at[pl.ds(row0, RB)], bufs[b], sems[b]).wait()
            pltpu.sync_copy(bufs[b], acc_out.at[idx2d.at[m]], add=True)
            pltpu.sync_copy(ones_v.at[:, pl.ds(0, HW)],
                            cnt_out.at[tgt2d.at[m], pl.ds(0, HW)], add=True)
        return 0

    lax.fori_loop(0, NBLK // 2, sbody, 0)


def _scatter_sc(idx2d_in, maskf2d, updated):
    mesh = plsc.VectorSubcoreMesh(core_axis_name="c", subcore_axis_name="s")
    fn = pl.kernel(
        _scatter_body,
        mesh=mesh,
        out_type=(
            jax.ShapeDtypeStruct((B * N_MAX, C), jnp.float32),
            jax.ShapeDtypeStruct((B * N_MAX + CNT_PAD, CW), jnp.float32),
        ),
        scratch_types=[
            pltpu.VMEM((NBLK, RB), jnp.int32),
            pltpu.VMEM((NBLK, RB), jnp.int32),
            pltpu.VMEM((NBLK, RB), jnp.float32),
            pltpu.VMEM((RB, C), jnp.float32),
            pltpu.VMEM((RB, C), jnp.float32),
            pltpu.VMEM((RB, CW), jnp.float32),
            pltpu.SemaphoreType.DMA,
            pltpu.SemaphoreType.DMA,
        ],
    )
    return fn(idx2d_in, maskf2d, updated)


def _combine_body(up_ref, acc_ref, cnt_ref, gamma_ref, out_ref):
    cnt = jnp.max(cnt_ref[...], axis=1, keepdims=True)
    denom = jnp.maximum(cnt, 1.0)
    out_ref[...] = up_ref[...] + acc_ref[...] * gamma_ref[...] / denom


def _combine_tc(up, acc, cnt, gamma2):
    BLK = 2048
    return pl.pallas_call(
        _combine_body,
        grid=(B * N_MAX // BLK,),
        in_specs=[
            pl.BlockSpec((BLK, C), lambda g: (g, 0)),
            pl.BlockSpec((BLK, C), lambda g: (g, 0)),
            pl.BlockSpec((BLK, 128), lambda g: (g, 0)),
        pl.BlockSpec((1, C), lambda g: (0, 0)),
        ],
        out_specs=pl.BlockSpec((BLK, C), lambda g: (g, 0)),
        out_shape=jax.ShapeDtypeStruct((B * N_MAX, C), jnp.float32),
    )(up, acc, cnt, gamma2)


def kernel(upscaled_feats, grouping_idx, grouping_point_mask, W_qkv, W_proj,
           b_proj, gamma):
    idx = jnp.where(grouping_idx < 0, 0, grouping_idx).astype(jnp.int32)
    idx_flat = idx.reshape(R)
    idx2d_in = idx.reshape(R // RB, RB)
    feats_flat = upscaled_feats.reshape(B * N_MAX, C)

    xg = _gather_sc(idx_flat, feats_flat)

    maskf = grouping_point_mask.astype(jnp.float32)
    mask_bias = jnp.tile((maskf.reshape(BG, 1, K) - 1.0) * 1e5, (1, 1, H))
    updated = _attn_tc(xg, mask_bias, W_qkv.astype(jnp.bfloat16),
                       W_proj.astype(jnp.bfloat16), b_proj.reshape(1, C))

    acc, cntp = _scatter_sc(idx2d_in, maskf.reshape(R // RB, RB), updated)

    out = _combine_tc(feats_flat, acc, cntp, gamma.reshape(1, C))
    return out.reshape(B, N_MAX, C)
